# Initial kernel scaffold; baseline (speedup 1.0000x reference)
import jax
import jax.numpy as jnp
from jax.experimental import pallas as pl


def kernel(features_list, W_edge, b_edge, W1, b1, W2, b2):
    raise NotImplementedError("write your pallas kernel here")


# speedup vs baseline: 5.3314x; 5.3314x over previous
import jax
import jax.numpy as jnp
from jax.experimental import pallas as pl
import jax.experimental.pallas.tpu as pltpu

N = 8
D = 128
BB = 512


def _fusion_kernel(x_ref, wsrc_ref, wdst_ref, be_ref, w1_ref, b1_ref,
                   w2_ref, b2_ref, out_ref):
    x = x_ref[:]
    wsrc = wsrc_ref[0, :]
    wdst = wdst_ref[0, :]
    be = be_ref[0, 0]

    a = jnp.sum(x * wsrc[None, None, :], axis=-1)
    c = jnp.sum(x * wdst[None, None, :], axis=-1)
    logits = a[:, None, :] + c[None, :, :] + be
    ew = jax.nn.sigmoid(logits)
    eye = jnp.eye(N, dtype=jnp.float32)[:, :, None]
    ew = ew * (1.0 - eye)

    deg = 1.0 + jnp.sum(ew, axis=0)
    inv_sqrt = jax.lax.rsqrt(deg)
    inv_deg = 1.0 / deg
    A = (ew * inv_sqrt[:, None, :] * inv_sqrt[None, :, :]
         + eye * inv_deg[None, :, :])

    def gcn(xv, W, b):
        xw = (jnp.dot(xv.reshape(N * BB, D), W[:, :],
                      preferred_element_type=jnp.float32)
              + b[0, :][None, :]).reshape(N, BB, D)
        outs = []
        for j in range(N):
            acc = A[0, j][:, None] * xw[0]
            for i in range(1, N):
                acc = acc + A[i, j][:, None] * xw[i]
            outs.append(acc)
        return outs

    h = [jax.nn.relu(o) for o in gcn(x, w1_ref, b1_ref)]
    out2 = gcn(jnp.stack(h, axis=0), w2_ref, b2_ref)
    out_ref[:] = jnp.stack(out2, axis=1)


def kernel(features_list, W_edge, b_edge, W1, b1, W2, b2):
    B = features_list.shape[1]
    wsrc = W_edge[:D, 0].reshape(1, D)
    wdst = W_edge[D:, 0].reshape(1, D)
    be = b_edge.reshape(1, 1)
    b1r = b1.reshape(1, D)
    b2r = b2.reshape(1, D)

    grid = (B // BB,)
    rep2 = lambda i: (0, 0)
    out = pl.pallas_call(
        _fusion_kernel,
        grid=grid,
        in_specs=[
            pl.BlockSpec((N, BB, D), lambda i: (0, i, 0)),
            pl.BlockSpec((1, D), rep2),
            pl.BlockSpec((1, D), rep2),
            pl.BlockSpec((1, 1), rep2),
            pl.BlockSpec((D, D), rep2),
            pl.BlockSpec((1, D), rep2),
            pl.BlockSpec((D, D), rep2),
            pl.BlockSpec((1, D), rep2),
        ],
        out_specs=pl.BlockSpec((BB, N, D), lambda i: (i, 0, 0)),
        out_shape=jax.ShapeDtypeStruct((B, N, D), jnp.float32),
        compiler_params=pltpu.CompilerParams(
            dimension_semantics=("parallel",),
        ),
    )(features_list, wsrc, wdst, be, W1, b1r, W2, b2r)
    return out
